# Initial kernel scaffold; baseline (speedup 1.0000x reference)
import jax
import jax.numpy as jnp
from jax.experimental import pallas as pl


def kernel(class_labels, table, W1, b1, W2, b2):
    raise NotImplementedError("write your pallas kernel here")


# speedup vs baseline: 1.2994x; 1.2994x over previous
import functools

import jax
import jax.numpy as jnp
from jax import lax
from jax.experimental import pallas as pl
from jax.experimental.pallas import tpu as pltpu
from jax.experimental.pallas import tpu_sc as plsc

_IDX_CHUNK = 128


def _gather_sc(table, idx):
  batch, dim = idx.shape[0], table.shape[1]
  info = plsc.get_sparse_core_info()
  num_workers = info.num_cores * info.num_subcores
  b_per_w = batch // num_workers
  n_chunks = b_per_w // _IDX_CHUNK
  mesh = plsc.VectorSubcoreMesh(core_axis_name="c", subcore_axis_name="s")

  @functools.partial(
      pl.kernel,
      mesh=mesh,
      out_type=jax.ShapeDtypeStruct((batch, dim), jnp.float32),
      scratch_types=[
          pltpu.VMEM((b_per_w,), jnp.int32),
          pltpu.VMEM((b_per_w, dim), jnp.float32),
          pltpu.SemaphoreType.DMA,
      ],
  )
  def gather_kernel(table_hbm, idx_hbm, out_hbm, idx_v, rows_v, sem):
    wid = lax.axis_index("s") * info.num_cores + lax.axis_index("c")
    base = wid * b_per_w
    pltpu.sync_copy(idx_hbm.at[pl.ds(base, b_per_w)], idx_v)
    copies = []
    for j in range(n_chunks):
      copies.append(
          pltpu.async_copy(
              table_hbm.at[idx_v.at[pl.ds(j * _IDX_CHUNK, _IDX_CHUNK)]],
              rows_v.at[pl.ds(j * _IDX_CHUNK, _IDX_CHUNK)],
              sem,
          )
      )
    for c in copies:
      c.wait()
    pltpu.sync_copy(rows_v, out_hbm.at[pl.ds(base, b_per_w)])

  return gather_kernel(table, idx)


def _mlp_body(emb_ref, w1_ref, b1_ref, w2_ref, b2_ref, out_ref):
  x = emb_ref[...]
  h = jnp.dot(x, w1_ref[...], preferred_element_type=jnp.float32) + b1_ref[...]
  h = h * jax.nn.sigmoid(h)
  out_ref[...] = (
      jnp.dot(h, w2_ref[...], preferred_element_type=jnp.float32) + b2_ref[...]
  )


def _mlp_tc(emb, w1t, b1, w2t, b2, blk=2048):
  batch, dim = emb.shape
  grid = (batch // blk,)
  return pl.pallas_call(
      _mlp_body,
      grid=grid,
      in_specs=[
          pl.BlockSpec((blk, dim), lambda i: (i, 0)),
          pl.BlockSpec((dim, dim), lambda i: (0, 0)),
          pl.BlockSpec((1, dim), lambda i: (0, 0)),
          pl.BlockSpec((dim, dim), lambda i: (0, 0)),
          pl.BlockSpec((1, dim), lambda i: (0, 0)),
      ],
      out_specs=pl.BlockSpec((blk, dim), lambda i: (i, 0)),
      out_shape=jax.ShapeDtypeStruct((batch, dim), jnp.float32),
  )(emb, w1t, b1, w2t, b2)


def kernel(class_labels, table, W1, b1, W2, b2):
  idx = class_labels.astype(jnp.int32)
  emb = _gather_sc(table, idx)
  return _mlp_tc(emb, W1.T, b1.reshape(1, -1), W2.T, b2.reshape(1, -1))
